# baseline (speedup 1.0000x reference)
import functools

import jax
import jax.numpy as jnp
from jax import lax
from jax.experimental import pallas as pl
from jax.experimental.pallas import tpu as pltpu
from jax.experimental.pallas import tpu_sc as plsc

VOCAB = 1000000
EMB = 64
BATCH = 16384
HIST = 50
NOUT = 2

NC = 2
NS = 16
NW = NC * NS
SPW = BATCH // NW
NCHUNK = 4
CHUNK = SPW // NCHUNK

BM = 2048
IDXW = 128


def _sums_body(idx_hbm, table_hbm, out_hbm, idx_sm, idx_v, acc_v,
               sem0, sem1, sem2, sem3):
    sems = (sem0, sem1, sem2, sem3)
    wid = lax.axis_index("s") * NC + lax.axis_index("c")
    base = wid * SPW

    pltpu.sync_copy(idx_hbm.at[pl.ds(base, SPW)], idx_sm)
    lane = lax.iota(jnp.int32, 16)

    def tr(j, carry):
        col = jnp.zeros((16,), jnp.int32) + j
        for g in range(SPW // 16):
            v = plsc.load_gather(idx_sm, [lane + g * 16, col])
            idx_v[pl.ds(j * SPW + g * 16, 16)] = v
        return carry

    lax.fori_loop(0, HIST, tr, 0)

    def chain(j, c):
        off = pl.multiple_of(j * SPW, SPW)
        src = table_hbm.at[idx_v.at[pl.ds(off + c * CHUNK, CHUNK)]]
        dst = acc_v.at[pl.ds(c * CHUNK, CHUNK)]
        return pltpu.make_async_copy(src, dst, sems[c])

    for c in range(NCHUNK):
        chain(0, c).start()

    def body(j, carry):
        for c in range(NCHUNK):
            d = chain(j, c)
            d.wait()
            d.start(add=True)
        return carry

    lax.fori_loop(1, HIST, body, 0)
    for c in range(NCHUNK):
        chain(HIST - 1, c).wait()

    pltpu.sync_copy(acc_v, out_hbm.at[pl.ds(base, SPW)])


def _linear_body(x_ref, wb_ref, o_ref):
    x = x_ref[...]
    wb = wb_ref[...]
    y = jnp.dot(x, wb[:EMB, :], preferred_element_type=jnp.float32)
    o_ref[...] = y * (1.0 / HIST) + wb[EMB, :][None, :]


@jax.jit
def _run(idx_r, wb, table):
    mesh = plsc.VectorSubcoreMesh(core_axis_name="c", subcore_axis_name="s")
    sums = pl.kernel(
        _sums_body,
        out_type=jax.ShapeDtypeStruct((BATCH, EMB), jnp.float32),
        mesh=mesh,
        compiler_params=pltpu.CompilerParams(
            needs_layout_passes=False, use_tc_tiling_on_sc=False),
        scratch_types=[
            pltpu.VMEM((SPW, IDXW), jnp.int32),
            pltpu.VMEM((HIST * SPW,), jnp.int32),
            pltpu.VMEM((SPW, EMB), jnp.float32),
            pltpu.SemaphoreType.DMA,
            pltpu.SemaphoreType.DMA,
            pltpu.SemaphoreType.DMA,
            pltpu.SemaphoreType.DMA,
        ],
    )(idx_r, table)

    return pl.pallas_call(
        _linear_body,
        out_shape=jax.ShapeDtypeStruct((BATCH, NOUT), jnp.float32),
        grid=(BATCH // BM,),
        in_specs=[
            pl.BlockSpec((BM, EMB), lambda i: (i, 0)),
            pl.BlockSpec((EMB + 1, NOUT), lambda i: (0, 0)),
        ],
        out_specs=pl.BlockSpec((BM, NOUT), lambda i: (i, 0)),
    )(sums, wb)


def kernel(inputs, table, W, b):
    idx_p = jnp.pad(inputs.astype(jnp.int32), ((0, 0), (0, IDXW - HIST)))
    wb = jnp.concatenate([W.T, b[None, :]], axis=0)
    return _run(idx_p, wb, table.astype(jnp.float32))


# speedup vs baseline: 1.0442x; 1.0442x over previous
import functools

import jax
import jax.numpy as jnp
from jax import lax
from jax.experimental import pallas as pl
from jax.experimental.pallas import tpu as pltpu
from jax.experimental.pallas import tpu_sc as plsc

VOCAB = 1000000
EMB = 64
BATCH = 16384
HIST = 50
NOUT = 2

NC = 2
NS = 16
NW = NC * NS
SPW = BATCH // NW
NCHUNK = 4
CHUNK = SPW // NCHUNK
OUTW = 128

BM = 2048


def _sums_body(idx_hbm, table_hbm, out_hbm, idx_v, acc_v,
               sem0, sem1, sem2, sem3):
    sems = (sem0, sem1, sem2, sem3)
    wid = lax.axis_index("s") * NC + lax.axis_index("c")
    base = wid * SPW

    pltpu.sync_copy(idx_hbm.at[wid], idx_v)

    def chain(j, c):
        off = pl.multiple_of(j * SPW, SPW)
        src = table_hbm.at[idx_v.at[pl.ds(off + c * CHUNK, CHUNK)]]
        dst = acc_v.at[pl.ds(c * CHUNK, CHUNK)]
        return pltpu.make_async_copy(src, dst, sems[c])

    for c in range(NCHUNK):
        chain(0, c).start()

    def body(j, carry):
        for c in range(NCHUNK):
            d = chain(j, c)
            d.wait()
            d.start(add=True)
        return carry

    lax.fori_loop(1, HIST, body, 0)
    for c in range(NCHUNK):
        chain(HIST - 1, c).wait()

    pltpu.sync_copy(acc_v, out_hbm.at[pl.ds(base, SPW), pl.ds(0, EMB)])


def _linear_body(x_ref, wb_ref, o_ref):
    x = x_ref[...]
    wb = wb_ref[...]
    y = jnp.dot(x[:, :EMB], wb[:EMB, :], preferred_element_type=jnp.float32)
    o_ref[...] = y * (1.0 / HIST) + wb[EMB, :][None, :]


@jax.jit
def _run(idx_r, wb, table):
    mesh = plsc.VectorSubcoreMesh(core_axis_name="c", subcore_axis_name="s")
    sums = pl.kernel(
        _sums_body,
        out_type=jax.ShapeDtypeStruct((BATCH, OUTW), jnp.float32),
        mesh=mesh,
        compiler_params=pltpu.CompilerParams(
            needs_layout_passes=False, use_tc_tiling_on_sc=False),
        scratch_types=[
            pltpu.VMEM((HIST * SPW,), jnp.int32),
            pltpu.VMEM((SPW, EMB), jnp.float32),
            pltpu.SemaphoreType.DMA,
            pltpu.SemaphoreType.DMA,
            pltpu.SemaphoreType.DMA,
            pltpu.SemaphoreType.DMA,
        ],
    )(idx_r, table)

    return pl.pallas_call(
        _linear_body,
        out_shape=jax.ShapeDtypeStruct((BATCH, NOUT), jnp.float32),
        grid=(BATCH // BM,),
        in_specs=[
            pl.BlockSpec((BM, OUTW), lambda i: (i, 0)),
            pl.BlockSpec((EMB + 1, NOUT), lambda i: (0, 0)),
        ],
        out_specs=pl.BlockSpec((BM, NOUT), lambda i: (i, 0)),
    )(sums, wb)


def kernel(inputs, table, W, b):
    idx_r = inputs.astype(jnp.int32).reshape(NW, SPW, HIST)
    idx_r = idx_r.transpose(0, 2, 1).reshape(NW, HIST * SPW)
    wb = jnp.concatenate([W.T, b[None, :]], axis=0)
    return _run(idx_r, wb, table.astype(jnp.float32))


# speedup vs baseline: 1.0443x; 1.0001x over previous
import functools

import jax
import jax.numpy as jnp
from jax import lax
from jax.experimental import pallas as pl
from jax.experimental.pallas import tpu as pltpu
from jax.experimental.pallas import tpu_sc as plsc

VOCAB = 1000000
EMB = 64
BATCH = 16384
HIST = 50
NOUT = 2

NC = 2
NS = 16
NW = NC * NS
SPW = BATCH // NW
NCHUNK = 4
CHUNK = SPW // NCHUNK
ROWS_W = HIST * NCHUNK
OUTW = 128

BM = 2048


def _idxprep_body(x_ref, o_ref):
    x = x_ref[...]
    y = jnp.transpose(x)
    o_ref[...] = y.reshape(ROWS_W, CHUNK)


def _sums_body(idx_hbm, table_hbm, out_hbm, idx_v, acc_v,
               sem0, sem1, sem2, sem3):
    sems = (sem0, sem1, sem2, sem3)
    wid = lax.axis_index("s") * NC + lax.axis_index("c")
    base = wid * SPW

    pltpu.sync_copy(idx_hbm.at[pl.ds(wid * ROWS_W, ROWS_W)], idx_v)

    def chain(j, c):
        src = table_hbm.at[idx_v.at[j * NCHUNK + c]]
        dst = acc_v.at[pl.ds(c * CHUNK, CHUNK)]
        return pltpu.make_async_copy(src, dst, sems[c])

    for c in range(NCHUNK):
        chain(0, c).start()

    def body(j, carry):
        for c in range(NCHUNK):
            d = chain(j, c)
            d.wait()
            d.start(add=True)
        return carry

    lax.fori_loop(1, HIST, body, 0)
    for c in range(NCHUNK):
        chain(HIST - 1, c).wait()

    pltpu.sync_copy(acc_v, out_hbm.at[pl.ds(base, SPW), pl.ds(0, EMB)])


def _linear_body(x_ref, wb_ref, o_ref):
    x = x_ref[...]
    wb = wb_ref[...]
    y = jnp.dot(x[:, :EMB], wb[:EMB, :], preferred_element_type=jnp.float32)
    o_ref[...] = y * (1.0 / HIST) + wb[EMB, :][None, :]


@jax.jit
def _run(idx, wb, table):
    idx_r = pl.pallas_call(
        _idxprep_body,
        out_shape=jax.ShapeDtypeStruct((NW * ROWS_W, CHUNK), jnp.int32),
        grid=(NW,),
        in_specs=[pl.BlockSpec((SPW, HIST), lambda w: (w, 0))],
        out_specs=pl.BlockSpec((ROWS_W, CHUNK), lambda w: (w, 0)),
    )(idx)

    mesh = plsc.VectorSubcoreMesh(core_axis_name="c", subcore_axis_name="s")
    sums = pl.kernel(
        _sums_body,
        out_type=jax.ShapeDtypeStruct((BATCH, OUTW), jnp.float32),
        mesh=mesh,
        compiler_params=pltpu.CompilerParams(
            needs_layout_passes=False, use_tc_tiling_on_sc=False),
        scratch_types=[
            pltpu.VMEM((ROWS_W, CHUNK), jnp.int32),
            pltpu.VMEM((SPW, EMB), jnp.float32),
            pltpu.SemaphoreType.DMA,
            pltpu.SemaphoreType.DMA,
            pltpu.SemaphoreType.DMA,
            pltpu.SemaphoreType.DMA,
        ],
    )(idx_r, table)

    return pl.pallas_call(
        _linear_body,
        out_shape=jax.ShapeDtypeStruct((BATCH, NOUT), jnp.float32),
        grid=(BATCH // BM,),
        in_specs=[
            pl.BlockSpec((BM, OUTW), lambda i: (i, 0)),
            pl.BlockSpec((EMB + 1, NOUT), lambda i: (0, 0)),
        ],
        out_specs=pl.BlockSpec((BM, NOUT), lambda i: (i, 0)),
    )(sums, wb)


def kernel(inputs, table, W, b):
    wb = jnp.concatenate([W.T, b[None, :]], axis=0)
    return _run(inputs.astype(jnp.int32), wb, table.astype(jnp.float32))
